# Initial kernel scaffold; baseline (speedup 1.0000x reference)
import jax
import jax.numpy as jnp
from jax.experimental import pallas as pl


def kernel(x, neurons, Wq, bq, Wk, bk, Wv, bv, Wp, bp, pq, gates, Wu, bu, Wd, bd, g1, beta1, g2, beta2):
    raise NotImplementedError("write your pallas kernel here")


# speedup vs baseline: 7.8559x; 7.8559x over previous
import functools
import math

import jax
import jax.numpy as jnp
from jax.experimental import pallas as pl

S, D = 2048, 768
H, DH = 12, 64
NN, K = 8192, 16
DFF = 3072
NPAT, KP = 64, 8

F32 = jnp.float32
BF16 = jnp.bfloat16


def _dot(a, b, dims=((1,), (0,))):
    return jax.lax.dot_general(a.astype(BF16), b.astype(BF16),
                               (dims, ((), ())),
                               preferred_element_type=F32)


def _rowmean(y):
    n = y.shape[1]
    ss = [jnp.sum(y[:, i * 128:(i + 1) * 128], axis=1, keepdims=True)
          for i in range(n // 128)]
    s = ((ss[0] + ss[1]) + (ss[2] + ss[3])) + (ss[4] + ss[5])
    return s * jnp.float32(1.0 / n)


def _ln(x, g, b):
    m = _rowmean(x)
    v = _rowmean((x - m) ** 2)
    return (x - m) / jnp.sqrt(v + 1e-5) * g + b


def _ln_qkv_kernel(x_ref, wq_ref, wk_ref, wv_ref, bqkv_ref, g_ref, b_ref,
                   normed_ref, q_ref, k_ref, v_ref):
    x = x_ref[...]
    normed = _ln(x, g_ref[...], b_ref[...])
    normed_ref[...] = normed
    nb = normed.astype(BF16)
    q_ref[...] = _dot(nb, wq_ref[...]) + bqkv_ref[0:1, :]
    k_ref[...] = _dot(nb, wk_ref[...]) + bqkv_ref[1:2, :]
    v_ref[...] = _dot(nb, wv_ref[...]) + bqkv_ref[2:3, :]


TB0 = 512


def _ln_qkv(x, wqT, wkT, wvT, bqkv, g, b):
    fs = jax.ShapeDtypeStruct((S, D), F32)
    tok = pl.BlockSpec((TB0, D), lambda i: (i, 0))
    full = lambda shape: pl.BlockSpec(shape, lambda i: (0,) * len(shape))
    return pl.pallas_call(
        _ln_qkv_kernel,
        grid=(S // TB0,),
        in_specs=[tok, full((D, D)), full((D, D)), full((D, D)),
                  full((3, D)), full((1, D)), full((1, D))],
        out_specs=(tok, tok, tok, tok),
        out_shape=(fs, fs, fs, fs),
    )(x, wqT, wkT, wvT, bqkv, g.reshape(1, D), b.reshape(1, D))


ATT_C = 1024


def _attn_kernel(q_ref, k_ref, v_ref, o_ref):
    q = q_ref[0]
    m = jnp.full((S, 1), -jnp.inf, dtype=F32)
    z = jnp.zeros((S, 1), dtype=F32)
    acc = jnp.zeros((S, DH), dtype=F32)
    for b in range(S // ATT_C):
        kb = k_ref[0][b * ATT_C:(b + 1) * ATT_C, :]
        vb = v_ref[0][b * ATT_C:(b + 1) * ATT_C, :]
        lb = _dot(q, kb, dims=((1,), (1,))) * (1.0 / math.sqrt(DH))
        mb = jnp.maximum(m, jnp.max(lb, axis=1, keepdims=True))
        corr = jnp.exp(m - mb)
        pb = jnp.exp(lb - mb)
        acc = acc * corr + _dot(pb, vb)
        z = z * corr + jnp.sum(pb, axis=1, keepdims=True)
        m = mb
    o_ref[0] = acc / z


def _attention(q, k, v):
    head = pl.BlockSpec((1, S, DH), lambda h: (h, 0, 0))
    return pl.pallas_call(
        _attn_kernel,
        grid=(H,),
        in_specs=[head, head, head],
        out_specs=head,
        out_shape=jax.ShapeDtypeStruct((H, S, DH), F32),
    )(q, k, v)


TB2 = 128


def _router_kernel(normed_ref, ctx_ref, neurons_ref, wpT_ref, bp_ref,
                   idx_ref, rout_ref):
    normed = normed_ref[...]
    ctx = ctx_ref[...]
    neurons_bf = neurons_ref[...]

    ts = _dot(normed, neurons_bf, dims=((1,), (1,)))
    cs = _dot(ctx, neurons_bf, dims=((1,), (1,)))

    combined = jnp.concatenate([normed, ctx], axis=1)
    l = _dot(combined, wpT_ref[...]) + bp_ref[...]
    m = jnp.max(l, axis=1, keepdims=True)
    e = jnp.exp(l - m)
    w = e / jnp.sum(e, axis=1, keepdims=True)
    scores = w[:, 0:1] * ts + w[:, 1:2] * cs

    iota = jax.lax.broadcasted_iota(jnp.int32, (TB2, NN), 1)
    prev = jnp.full((TB2, 1), jnp.inf, dtype=F32)
    vals, idxs = [], []
    for _ in range(K):
        cand = jnp.where(scores < prev, scores, -jnp.inf)
        vr = jnp.max(cand, axis=1, keepdims=True)
        ir = jnp.min(jnp.where(scores == vr, iota, NN), axis=1, keepdims=True)
        vals.append(vr)
        idxs.append(ir)
        prev = vr
    topv = jnp.concatenate(vals, axis=1)
    idx_ref[...] = jnp.concatenate(idxs, axis=1)

    v1 = topv[:, 0:1]
    thr = topv[:, K - 1:K]
    z = jnp.sum(jnp.exp(topv - v1), axis=1, keepdims=True)
    ew = jnp.where(scores >= thr, jnp.exp(scores - v1), 0.0)
    rout = _dot(ew, neurons_bf)
    rout_ref[...] = rout / z


def _router(normed, ctx, neurons_bf, wpT, bp):
    tok = pl.BlockSpec((TB2, D), lambda i: (i, 0))
    full = lambda shape: pl.BlockSpec(shape, lambda i: (0,) * len(shape))
    return pl.pallas_call(
        _router_kernel,
        grid=(S // TB2,),
        in_specs=[tok, tok, full((NN, D)), full((2 * D, 2)), full((1, 2))],
        out_specs=(pl.BlockSpec((TB2, K), lambda i: (i, 0)),
                   pl.BlockSpec((TB2, D), lambda i: (i, 0))),
        out_shape=(jax.ShapeDtypeStruct((S, K), jnp.int32),
                   jax.ShapeDtypeStruct((S, D), F32)),
    )(normed, ctx, neurons_bf, wpT, bp.reshape(1, 2))


TB3 = 256


def _ffn_kernel(x_ref, rout_ref, pq_ref, gates_ref, wu_ref, bu_ref,
                wd_ref, bd_ref, g2_ref, b2_ref, out_ref):
    x2 = x_ref[...] + rout_ref[...]
    normed2 = _ln(x2, g2_ref[...], b2_ref[...])

    ps = _dot(rout_ref[...], pq_ref[...],
              dims=((1,), (1,))) * (1.0 / math.sqrt(D))
    prev = jnp.full((TB3, 1), jnp.inf, dtype=F32)
    vals = []
    for _ in range(KP):
        cand = jnp.where(ps < prev, ps, -jnp.inf)
        vr = jnp.max(cand, axis=1, keepdims=True)
        vals.append(vr)
        prev = vr
    topv = jnp.concatenate(vals, axis=1)
    v1 = topv[:, 0:1]
    thr = topv[:, KP - 1:KP]
    z = jnp.sum(jnp.exp(topv - v1), axis=1, keepdims=True)
    ew = jnp.where(ps >= thr, jnp.exp(ps - v1), 0.0)
    gate = _dot(ew, gates_ref[...]) / z

    h = _dot(normed2, wu_ref[...], dims=((1,), (1,))) + bu_ref[...]
    h = h * (1.0 / (1.0 + jnp.exp(-gate)))
    h = 0.5 * h * (1.0 + jax.lax.erf(h * (1.0 / math.sqrt(2.0))))
    out_ref[...] = x2 + _dot(h, wd_ref[...], dims=((1,), (1,))) + bd_ref[...]


def _ffn(x, rout, pq, gates, wu, bu, wd, bd, g2, b2):
    tok = pl.BlockSpec((TB3, D), lambda i: (i, 0))
    full = lambda shape: pl.BlockSpec(shape, lambda i: (0,) * len(shape))
    return pl.pallas_call(
        _ffn_kernel,
        grid=(S // TB3,),
        in_specs=[tok, tok, full((NPAT, D)), full((NPAT, DFF)),
                  full((DFF, D)), full((1, DFF)), full((D, DFF)),
                  full((1, D)), full((1, D)), full((1, D))],
        out_specs=tok,
        out_shape=jax.ShapeDtypeStruct((S, D), F32),
    )(x, rout, pq.astype(BF16), gates.astype(BF16),
      wu.astype(BF16), bu.reshape(1, DFF), wd.astype(BF16),
      bd.reshape(1, D), g2.reshape(1, D), b2.reshape(1, D))


def kernel(x, neurons, Wq, bq, Wk, bk, Wv, bv, Wp, bp, pq, gates,
           Wu, bu, Wd, bd, g1, beta1, g2, beta2):
    x2d = x.reshape(S, D)
    bqkv = jnp.stack([bq, bk, bv], axis=0)
    normed, q, k, v = _ln_qkv(x2d, Wq.T.astype(BF16), Wk.T.astype(BF16),
                              Wv.T.astype(BF16), bqkv, g1, beta1)
    hm = lambda a: a.reshape(S, H, DH).transpose(1, 0, 2)
    ctx = _attention(hm(q), hm(k), hm(v))
    ctx = ctx.transpose(1, 0, 2).reshape(S, D)
    topk_idx, rout = _router(normed, ctx, neurons.astype(BF16),
                             Wp.T.astype(BF16), bp)
    out = _ffn(x2d, rout, pq, gates, Wu, bu, Wd, bd, g2, beta2)
    return out.reshape(1, S, D), topk_idx.reshape(1, S, K)
